# Initial kernel scaffold; baseline (speedup 1.0000x reference)
import jax
import jax.numpy as jnp
from jax.experimental import pallas as pl


def kernel(data):
    raise NotImplementedError("write your pallas kernel here")


# speedup vs baseline: 1.9543x; 1.9543x over previous
import functools

import numpy as np
import jax
import jax.numpy as jnp
from jax import lax
from jax.experimental import pallas as pl
from jax.experimental.pallas import tpu as pltpu
from jax.experimental.pallas import tpu_sc as plsc

T = 4096
C = 1086
CP = 1152
OUT_T = 256
OUT_C = 454
BLK = 512
GRID = T // BLK
NW = 32
ROWS_PER = OUT_T // NW

_FACE = np.array([0, 6, 7, 11, 12, 13, 14, 15, 17, 22, 23, 24, 25, 26, 30, 31,
    33, 37, 38, 39, 40, 41, 42, 56, 61, 62, 72, 73, 74, 76, 77, 78, 80, 81, 82,
    84, 86, 87, 88, 89, 90, 91, 95, 96, 110, 112, 113, 122, 128, 130, 133, 144,
    145, 146, 153, 154, 155, 157, 158, 159, 160, 161, 163, 168, 173, 178, 179,
    180, 181, 183, 184, 185, 188, 189, 190, 191, 193, 196, 197, 232, 233, 243,
    244, 245, 246, 247, 249, 252, 253, 254, 255, 256, 259, 260, 263, 267, 268,
    269, 270, 271, 272, 286, 291, 292, 302, 303, 304, 306, 307, 308, 310, 311,
    312, 314, 316, 317, 318, 319, 320, 321, 324, 325, 339, 341, 351, 357, 359,
    362, 373, 374, 375, 380, 381, 382, 384, 385, 386, 387, 388, 390, 398, 402,
    403, 404, 405, 407, 408, 409, 412, 413, 414, 415, 417, 419, 453, 463, 464,
    465, 466, 467], dtype=np.int32)
_POSE = np.arange(489, 514, dtype=np.int32)
_LH = np.arange(468, 489, dtype=np.int32)
_RH = np.arange(522, 543, dtype=np.int32)
_HANDS = np.concatenate([_LH, _RH])
_LIDX = np.concatenate([_FACE, _POSE, _LH, _RH])

_PART_BOUNDS = [0, 160, 181, 206, 227]
_PART_LEN = [160, 21, 25, 21]


def _build_consts():
    P = np.zeros((CP, OUT_C), np.float32)
    for u, li in enumerate(_LIDX):
        for ch in (0, 1):
            P[2 * li + ch, 2 * u + ch] = 1.0
    W8 = np.zeros((C, 8), np.float32)
    for p in range(4):
        for li in _LIDX[_PART_BOUNDS[p]:_PART_BOUNDS[p + 1]]:
            for ch in (0, 1):
                W8[2 * li + ch, 2 * p + ch] = 1.0
    S8 = np.zeros((8, OUT_C), np.float32)
    for u in range(227):
        p = next(k for k in range(4)
                 if _PART_BOUNDS[k] <= u < _PART_BOUNDS[k + 1])
        for ch in (0, 1):
            S8[2 * p + ch, 2 * u + ch] = 1.0
    NV = np.array([float(T) * _PART_LEN[p] for p in range(4) for _ in (0, 1)],
                  np.float32).reshape(1, 8)
    WH = np.zeros((C, 1), np.float32)
    for i in _HANDS:
        WH[2 * i, 0] = 1.0
        WH[2 * i + 1, 0] = 1.0
    LT = np.tril(np.ones((BLK, BLK), np.float32))
    return P, W8, S8, NV, WH, LT


_P, _W8, _S8, _NV, _WH, _LT = _build_consts()


def _dot(a, b):
    return lax.dot_general(a, b, (((1,), (0,)), ((), ())),
                           precision=lax.Precision.HIGHEST,
                           preferred_element_type=jnp.float32)


def _reduce_body(x_ref, wh_ref, lt_ref, colsum_ref, colsq_ref, cum_ref,
                 xpad_ref, carry_ref):
    g = pl.program_id(0)
    x = x_ref[...]
    xpad_ref[...] = jnp.concatenate(
        [x, jnp.zeros((BLK, CP - C), jnp.float32)], axis=1)
    hs = _dot(x, wh_ref[...])
    m = (hs > 0.0).astype(jnp.float32)
    cumb = _dot(lt_ref[...], m)

    @pl.when(g == 0)
    def _():
        carry_ref[0] = 0.0

    carry = carry_ref[0]
    cum_ref[...] = cumb + carry
    carry_ref[0] = carry + jnp.sum(m)

    xm = x * m
    s = jnp.sum(xm, axis=0, keepdims=True)
    q = jnp.sum(xm * x, axis=0, keepdims=True)

    @pl.when(g == 0)
    def _():
        colsum_ref[...] = s
        colsq_ref[...] = q

    @pl.when(g != 0)
    def _():
        colsum_ref[...] = colsum_ref[...] + s
        colsq_ref[...] = colsq_ref[...] + q


_reduce = pl.pallas_call(
    _reduce_body,
    grid=(GRID,),
    in_specs=[
        pl.BlockSpec((BLK, C), lambda g: (g, 0)),
        pl.BlockSpec((C, 1), lambda g: (0, 0)),
        pl.BlockSpec((BLK, BLK), lambda g: (0, 0)),
    ],
    out_specs=[
        pl.BlockSpec((1, C), lambda g: (0, 0)),
        pl.BlockSpec((1, C), lambda g: (0, 0)),
        pl.BlockSpec((BLK, 1), lambda g: (g, 0)),
        pl.BlockSpec((BLK, CP), lambda g: (g, 0)),
    ],
    out_shape=[
        jax.ShapeDtypeStruct((1, C), jnp.float32),
        jax.ShapeDtypeStruct((1, C), jnp.float32),
        jax.ShapeDtypeStruct((T, 1), jnp.float32),
        jax.ShapeDtypeStruct((T, CP), jnp.float32),
    ],
    scratch_shapes=[pltpu.SMEM((1,), jnp.float32)],
    compiler_params=pltpu.CompilerParams(
        dimension_semantics=("arbitrary",)),
)


@functools.cache
def _get_sc_route_gather():
    mesh = plsc.VectorSubcoreMesh(core_axis_name="c", subcore_axis_name="s")

    @functools.partial(
        pl.kernel,
        mesh=mesh,
        out_type=jax.ShapeDtypeStruct((OUT_T, CP), jnp.float32),
        scratch_types=[
            pltpu.VMEM((T,), jnp.float32),
            pltpu.VMEM((16,), jnp.int32),
            pltpu.VMEM((ROWS_PER, CP), jnp.float32),
            pltpu.SemaphoreType.DMA,
        ],
        compiler_params=pltpu.CompilerParams(needs_layout_passes=False),
    )
    def _sc_route_gather(cum_hbm, data_hbm, rows_out, cum_v, idx_v, rows_v,
                         sem):
        w = lax.axis_index("s") * 2 + lax.axis_index("c")
        pltpu.sync_copy(cum_hbm, cum_v)
        tail = cum_v[pl.ds(T - 16, 16)]
        total = tail[15]
        lane = lax.iota(jnp.int32, 16)
        j = lane & 7
        r1 = ((w * ROWS_PER + j) * 16 + 9).astype(jnp.float32)
        pos = jnp.zeros((16,), jnp.int32)
        for step in (2048, 1024, 512, 256, 128, 64, 32, 16, 8, 4, 2, 1):
            v = plsc.load_gather(cum_v, [pos + (step - 1)])
            pos = jnp.where(v < r1, pos + step, pos)
        posf = jnp.where(jnp.full((16,), total) >= r1, pos, 0)
        idx_v[...] = posf
        pltpu.async_copy(data_hbm.at[idx_v.at[pl.ds(0, 8)]], rows_v,
                         sem).wait()
        pltpu.sync_copy(rows_v, rows_out.at[pl.ds(w * ROWS_PER, ROWS_PER)])

    return _sc_route_gather


def _final_body(rows_ref, colsum_ref, colsq_ref, row0_ref, cnt_ref,
                w8_ref, s8_ref, p_ref, nv_ref, out_ref):
    fill = jnp.float32(T) - cnt_ref[0, 0]
    row0 = row0_ref[...]
    cs = colsum_ref[...] + fill * row0
    cq = colsq_ref[...] + fill * row0 * row0
    psum = _dot(cs, w8_ref[...])
    psq = _dot(cq, w8_ref[...])
    n = nv_ref[...]
    mean = psum / n
    var = jnp.maximum(psq / n - mean * mean, 0.0)
    std = jnp.sqrt(var)
    meanv = _dot(mean, s8_ref[...])
    stdv = _dot(std, s8_ref[...])
    sel = _dot(rows_ref[...], p_ref[...])
    o = jnp.where(sel == 0.0, 0.0, (sel - meanv) / stdv)
    o = jnp.where(jnp.isnan(o), 0.0, o)
    out_ref[...] = o


_final = pl.pallas_call(
    _final_body,
    out_shape=jax.ShapeDtypeStruct((OUT_T, OUT_C), jnp.float32),
)


def kernel(data):
    x = data.reshape(T, C)
    colsum, colsq, cum, xpad = _reduce(x, jnp.asarray(_WH), jnp.asarray(_LT))
    rows = _get_sc_route_gather()(cum.reshape(T), xpad)
    out = _final(rows, colsum, colsq, x[0:1], cum[T - 1:T],
                 jnp.asarray(_W8), jnp.asarray(_S8), jnp.asarray(_P),
                 jnp.asarray(_NV))
    return out, jnp.arange(OUT_T, dtype=jnp.float32)
